# Initial kernel scaffold; baseline (speedup 1.0000x reference)
import jax
import jax.numpy as jnp
from jax.experimental import pallas as pl


def kernel(x, edge_index, batch_vec, u_idx, v_idx, conv_w0, conv_b0, conv_w1, conv_b1, conv_w2, conv_b2, mlp_w1, mlp_b1, mlp_w2, mlp_b2):
    raise NotImplementedError("write your pallas kernel here")


# speedup vs baseline: 8.2466x; 8.2466x over previous
import functools

import jax
import jax.numpy as jnp
from jax import lax
from jax.experimental import pallas as pl
from jax.experimental.pallas import tpu as pltpu
from jax.experimental.pallas import tpu_sc as plsc

_NC = 2
_NS = 16
_NW = _NC * _NS
_CH = 128
_ROWB = 256


def _rup(x, m):
    return (x + m - 1) // m * m


def _sc_scatter(src_i, dst_i, table, ar):
    et = src_i.shape[0]
    d = table.shape[1]
    epw = et // _NW
    steps = epw // _CH
    zr = ar // _NS
    zfull, ztail = zr // _CH, zr % _CH
    mesh = plsc.VectorSubcoreMesh(core_axis_name="c", subcore_axis_name="s")

    @functools.partial(
        pl.kernel,
        out_type=jax.ShapeDtypeStruct((_NC, ar, d), jnp.float32),
        mesh=mesh,
        scratch_types=[
            pltpu.VMEM((_CH,), jnp.int32),
            pltpu.VMEM((_CH,), jnp.int32),
            pltpu.VMEM((_CH, d), jnp.float32),
            pltpu.VMEM_SHARED((ar, d), jnp.float32),
            pltpu.SemaphoreType.DMA,
        ],
    )
    def k(zeros_hbm, src_hbm, dst_hbm, table_hbm, out_hbm,
          idx_s, idx_d, rows, acc, sem):
        c = lax.axis_index("c")
        s = lax.axis_index("s")
        wid = s * _NC + c
        pltpu.sync_copy(zeros_hbm, rows)
        zb = s * zr
        for j in range(zfull):
            pltpu.sync_copy(rows, acc.at[pl.ds(zb + j * _CH, _CH)])
        if ztail:
            pltpu.sync_copy(rows.at[pl.ds(0, ztail)],
                            acc.at[pl.ds(zb + zfull * _CH, ztail)])
        plsc.subcore_barrier()

        base = wid * epw

        def step(i, carry):
            off = base + i * _CH
            pltpu.sync_copy(src_hbm.at[pl.ds(off, _CH)], idx_s)
            pltpu.async_copy(table_hbm.at[idx_s], rows, sem).wait()
            pltpu.sync_copy(dst_hbm.at[pl.ds(off, _CH)], idx_d)
            pltpu.sync_copy(rows, acc.at[idx_d], add=True)
            return carry

        lax.fori_loop(0, steps, step, 0)
        plsc.subcore_barrier()

        for cc in range(_NC):
            @pl.when(c == cc)
            def _():
                for j in range(zfull):
                    pltpu.sync_copy(acc.at[pl.ds(zb + j * _CH, _CH)], rows)
                    pltpu.sync_copy(rows, out_hbm.at[cc, pl.ds(zb + j * _CH, _CH)])
                if ztail:
                    pltpu.sync_copy(acc.at[pl.ds(zb + zfull * _CH, ztail)],
                                    rows.at[pl.ds(0, ztail)])
                    pltpu.sync_copy(rows.at[pl.ds(0, ztail)],
                                    out_hbm.at[cc, pl.ds(zb + zfull * _CH, ztail)])

    zeros = jnp.zeros((_CH, d), jnp.float32)
    return k(zeros, src_i, dst_i, table)


def _sc_count(dst_i, ar):
    et = dst_i.shape[0]
    d = 128
    epw = et // _NW
    steps = epw // _CH
    zr = ar // _NS
    zfull, ztail = zr // _CH, zr % _CH
    mesh = plsc.VectorSubcoreMesh(core_axis_name="c", subcore_axis_name="s")

    @functools.partial(
        pl.kernel,
        out_type=jax.ShapeDtypeStruct((_NC, ar, d), jnp.float32),
        mesh=mesh,
        scratch_types=[
            pltpu.VMEM((_CH,), jnp.int32),
            pltpu.VMEM((_CH, d), jnp.float32),
            pltpu.VMEM_SHARED((ar, d), jnp.float32),
        ],
    )
    def k(zeros_hbm, ones_hbm, dst_hbm, out_hbm, idx_d, rows, acc):
        c = lax.axis_index("c")
        s = lax.axis_index("s")
        wid = s * _NC + c
        pltpu.sync_copy(zeros_hbm, rows)
        zb = s * zr
        for j in range(zfull):
            pltpu.sync_copy(rows, acc.at[pl.ds(zb + j * _CH, _CH)])
        if ztail:
            pltpu.sync_copy(rows.at[pl.ds(0, ztail)],
                            acc.at[pl.ds(zb + zfull * _CH, ztail)])
        plsc.subcore_barrier()

        pltpu.sync_copy(ones_hbm, rows)
        base = wid * epw

        def step(i, carry):
            off = base + i * _CH
            pltpu.sync_copy(dst_hbm.at[pl.ds(off, _CH)], idx_d)
            pltpu.sync_copy(rows, acc.at[idx_d], add=True)
            return carry

        lax.fori_loop(0, steps, step, 0)
        plsc.subcore_barrier()

        for cc in range(_NC):
            @pl.when(c == cc)
            def _():
                for j in range(zfull):
                    pltpu.sync_copy(acc.at[pl.ds(zb + j * _CH, _CH)], rows)
                    pltpu.sync_copy(rows, out_hbm.at[cc, pl.ds(zb + j * _CH, _CH)])
                if ztail:
                    pltpu.sync_copy(acc.at[pl.ds(zb + zfull * _CH, ztail)],
                                    rows.at[pl.ds(0, ztail)])
                    pltpu.sync_copy(rows.at[pl.ds(0, ztail)],
                                    out_hbm.at[cc, pl.ds(zb + zfull * _CH, ztail)])

    zeros = jnp.zeros((_CH, d), jnp.float32)
    ones = jnp.ones((_CH, d), jnp.float32)
    return k(zeros, ones, dst_i)


def _tc_prep(dp, xp, w0, n, np_):
    in_ch = xp.shape[1]
    hid = w0.shape[1]

    def body(dp_ref, x_ref, w0_ref, hh_ref, dinv_ref):
        i = pl.program_id(0)
        deg = dp_ref[0] + dp_ref[1]
        degc = deg[:, 0:1] + 1.0
        rows = i * _ROWB + lax.broadcasted_iota(jnp.int32, (_ROWB, 1), 0)
        dinv = jnp.where(rows < n, lax.rsqrt(degc), 0.0)
        hh_ref[...] = jnp.dot(x_ref[...], w0_ref[...],
                              preferred_element_type=jnp.float32) * dinv
        dinv_ref[...] = dinv

    return pl.pallas_call(
        body,
        grid=(np_ // _ROWB,),
        in_specs=[
            pl.BlockSpec((2, _ROWB, 128), lambda i: (0, i, 0)),
            pl.BlockSpec((_ROWB, in_ch), lambda i: (i, 0)),
            pl.BlockSpec((in_ch, hid), lambda i: (0, 0)),
        ],
        out_specs=[
            pl.BlockSpec((_ROWB, hid), lambda i: (i, 0)),
            pl.BlockSpec((_ROWB, 1), lambda i: (i, 0)),
        ],
        out_shape=[
            jax.ShapeDtypeStruct((np_, hid), jnp.float32),
            jax.ShapeDtypeStruct((np_, 1), jnp.float32),
        ],
    )(dp, xp, w0)


def _tc_layer(p, hh, dinv, bias, w):
    np_, hid = hh.shape

    def body(p_ref, hh_ref, dinv_ref, b_ref, w_ref, o_ref):
        dv = dinv_ref[...]
        h = dv * (p_ref[0] + p_ref[1] + hh_ref[...]) + b_ref[...]
        h = jnp.maximum(h, 0.0)
        o_ref[...] = jnp.dot(h, w_ref[...],
                             preferred_element_type=jnp.float32) * dv

    return pl.pallas_call(
        body,
        grid=(np_ // _ROWB,),
        in_specs=[
            pl.BlockSpec((2, _ROWB, hid), lambda i: (0, i, 0)),
            pl.BlockSpec((_ROWB, hid), lambda i: (i, 0)),
            pl.BlockSpec((_ROWB, 1), lambda i: (i, 0)),
            pl.BlockSpec((1, hid), lambda i: (0, 0)),
            pl.BlockSpec((hid, hid), lambda i: (0, 0)),
        ],
        out_specs=pl.BlockSpec((_ROWB, hid), lambda i: (i, 0)),
        out_shape=jax.ShapeDtypeStruct((np_, hid), jnp.float32),
    )(p, hh, dinv, bias, w)


def _tc_combine(p, hh, dinv, bias):
    np_, hid = hh.shape

    def body(p_ref, hh_ref, dinv_ref, b_ref, o_ref):
        o_ref[...] = (dinv_ref[...] * (p_ref[0] + p_ref[1] + hh_ref[...])
                      + b_ref[...])

    return pl.pallas_call(
        body,
        grid=(np_ // _ROWB,),
        in_specs=[
            pl.BlockSpec((2, _ROWB, hid), lambda i: (0, i, 0)),
            pl.BlockSpec((_ROWB, hid), lambda i: (i, 0)),
            pl.BlockSpec((_ROWB, 1), lambda i: (i, 0)),
            pl.BlockSpec((1, hid), lambda i: (0, 0)),
        ],
        out_specs=pl.BlockSpec((_ROWB, hid), lambda i: (i, 0)),
        out_shape=jax.ShapeDtypeStruct((np_, hid), jnp.float32),
    )(p, hh, dinv, bias)


def _tc_head(pp, cp, w1, b1, w2, b2, b, np_, hid):
    pl_ar = pp.shape[1]

    def body(pp_ref, cp_ref, w1_ref, b1_ref, w2_ref, b2_ref, o_ref):
        ps = pp_ref[0] + pp_ref[1]
        cnt = (cp_ref[0] + cp_ref[1])[:, 0:1]
        g = ps[0:b] / jnp.maximum(cnt, 1.0)
        hu = ps[b:2 * b]
        hv = ps[2 * b:3 * b]
        hid_a = (jnp.dot(g, w1_ref[0:hid],
                         preferred_element_type=jnp.float32)
                 + jnp.dot(hu, w1_ref[hid:2 * hid],
                           preferred_element_type=jnp.float32)
                 + jnp.dot(hv, w1_ref[2 * hid:3 * hid],
                           preferred_element_type=jnp.float32)
                 + b1_ref[...])
        hid_a = jnp.maximum(hid_a, 0.0)
        o_ref[...] = jnp.dot(hid_a, w2_ref[...],
                             preferred_element_type=jnp.float32) + b2_ref[...]

    return pl.pallas_call(
        body,
        grid=(1,),
        in_specs=[
            pl.BlockSpec((2, 3 * b, hid), lambda i: (0, 0, 0)),
            pl.BlockSpec((2, b, 128), lambda i: (0, np_ // b, 0)),
            pl.BlockSpec((3 * hid, hid), lambda i: (0, 0)),
            pl.BlockSpec((1, hid), lambda i: (0, 0)),
            pl.BlockSpec((hid, 1), lambda i: (0, 0)),
            pl.BlockSpec((1, 1), lambda i: (0, 0)),
        ],
        out_specs=pl.BlockSpec((b, 1), lambda i: (0, 0)),
        out_shape=jax.ShapeDtypeStruct((b, 1), jnp.float32),
    )(pp, cp, w1, b1, w2, b2)


def kernel(x, edge_index, batch_vec, u_idx, v_idx,
           conv_w0, conv_b0, conv_w1, conv_b1, conv_w2, conv_b2,
           mlp_w1, mlp_b1, mlp_w2, mlp_b2):
    n, in_ch = x.shape
    e = edge_index.shape[1]
    b = u_idx.shape[0]
    hid = conv_w0.shape[1]

    np_ = _rup(n + 1, 1024)
    ep = _rup(e, _NW * _CH)
    ct_e = _rup(ep + n, _NW * _CH)
    ct_ar = _rup(np_ + b + 1, _CH)
    pe = _rup(n + 2 * b, _NW * _CH)
    pl_ar = _rup(3 * b + 1, _CH)

    i32 = jnp.int32
    src_p = jnp.concatenate([edge_index[0], jnp.full((ep - e,), n, i32)])
    dst_p = jnp.concatenate([edge_index[1], jnp.full((ep - e,), n, i32)])
    cnt_idx = jnp.concatenate([
        dst_p, np_ + batch_vec, jnp.full((ct_e - ep - n,), np_ + b, i32)])
    pool_src = jnp.concatenate([
        jnp.arange(n, dtype=i32), u_idx, v_idx,
        jnp.full((pe - n - 2 * b,), n, i32)])
    pool_dst = jnp.concatenate([
        batch_vec, b + jnp.arange(b, dtype=i32), 2 * b + jnp.arange(b, dtype=i32),
        jnp.full((pe - n - 2 * b,), 3 * b, i32)])
    xp = jnp.pad(x, ((0, np_ - n), (0, 0)))

    cp = _sc_count(cnt_idx, ct_ar)

    hh0, dinv = _tc_prep(cp, xp, conv_w0, n, np_)

    p0 = _sc_scatter(src_p, dst_p, hh0, np_)
    hh1 = _tc_layer(p0, hh0, dinv, conv_b0.reshape(1, hid), conv_w1)
    p1 = _sc_scatter(src_p, dst_p, hh1, np_)
    hh2 = _tc_layer(p1, hh1, dinv, conv_b1.reshape(1, hid), conv_w2)
    p2 = _sc_scatter(src_p, dst_p, hh2, np_)
    h3 = _tc_combine(p2, hh2, dinv, conv_b2.reshape(1, hid))

    pp = _sc_scatter(pool_src, pool_dst, h3, pl_ar)

    out = _tc_head(pp, cp, mlp_w1, mlp_b1.reshape(1, hid),
                   mlp_w2, mlp_b2.reshape(1, 1), b, np_, hid)
    return out[:, 0]
